# baseline (speedup 1.0000x reference)
import jax
import jax.numpy as jnp
from jax.experimental import pallas as pl
from jax.experimental.pallas import tpu as pltpu
from jax.experimental.pallas import tpu_sc as plsc

E = 64
TOP_K = 8
D_MODEL = 1024
D_FF = 512
N_GROUP = 8
TOPK_GROUP = 4
T = 2048
GS = E // N_GROUP
M = T * TOP_K
TM = 256
NT = M // TM + E
M_PAD = NT * TM


def _routing_kernel(x_ref, gw_ref, pos_ref, wk_ref, counts_ref):
    x = x_ref[...]
    gw = gw_ref[...]
    logits = jax.lax.dot_general(x, gw, (((1,), (1,)), ((), ())),
                                 preferred_element_type=jnp.float32)
    m = jnp.max(logits, axis=1, keepdims=True)
    ex = jnp.exp(logits - m)
    scores = ex / jnp.sum(ex, axis=1, keepdims=True)

    lane = jax.lax.broadcasted_iota(jnp.int32, (T, E), 1)
    group_of_lane = lane // GS

    G = jnp.zeros((T, E), jnp.float32)
    gmaxes = []
    for g in range(N_GROUP):
        gm = jnp.max(jnp.where(group_of_lane == g, scores, -jnp.inf), axis=1,
                     keepdims=True)
        gmaxes.append(gm)
        G = jnp.where(group_of_lane == g, gm, G)

    rank = jnp.zeros((T, E), jnp.int32)
    for g in range(N_GROUP):
        gm = gmaxes[g]
        rank = rank + jnp.where(gm > G, 1, 0) \
                    + jnp.where((gm == G) & (g < group_of_lane), 1, 0)
    ms = jnp.where(rank < TOPK_GROUP, scores, 0.0)

    work = ms
    denom = jnp.zeros((T, 1), jnp.float32)
    picks = []
    mxs = []
    for _ in range(TOP_K):
        mx = jnp.max(work, axis=1, keepdims=True)
        pick_lane = jnp.min(jnp.where(work == mx, lane, E), axis=1,
                            keepdims=True)
        pick = lane == pick_lane
        picks.append(pick)
        mxs.append(mx)
        denom = denom + mx
        work = jnp.where(pick, -1.0, work)

    chosen = picks[0]
    for p in picks[1:]:
        chosen = chosen | p
    c32 = chosen.astype(jnp.int32)

    inc = c32
    s = 1
    while s < T:
        shifted = jnp.concatenate(
            [jnp.zeros((s, E), jnp.int32), inc[: T - s, :]], axis=0)
        inc = inc + shifted
        s *= 2
    col_excl = inc - c32

    counts = jnp.sum(c32, axis=0, keepdims=True)
    counts_ref[...] = counts

    pc = ((counts + (TM - 1)) // TM) * TM
    off = pc
    s = 1
    while s < E:
        off_sh = jnp.concatenate(
            [jnp.zeros((1, s), jnp.int32), off[:, : E - s]], axis=1)
        off = off + off_sh
        s *= 2
    offsets = off - pc

    posmat = offsets + col_excl

    inv_denom = 1.0 / (denom + 1e-20)
    pos_cols = []
    wk_cols = []
    for k in range(TOP_K):
        pos_cols.append(jnp.sum(jnp.where(picks[k], posmat, 0), axis=1,
                                keepdims=True))
        wk_cols.append(mxs[k] * inv_denom)
    pos_ref[...] = jnp.concatenate(pos_cols, axis=1)
    wk_ref[...] = jnp.concatenate(wk_cols, axis=1)


def _gmm_kernel(te_ref, xs_ref, w1_ref, w3_ref, w2_ref, ys_ref):
    x = xs_ref[...].astype(jnp.bfloat16)
    w1e = w1_ref[0].astype(jnp.bfloat16)
    w3e = w3_ref[0].astype(jnp.bfloat16)
    w2e = w2_ref[0].astype(jnp.bfloat16)
    h1 = jax.lax.dot_general(x, w1e, (((1,), (1,)), ((), ())),
                             preferred_element_type=jnp.float32)
    h3 = jax.lax.dot_general(x, w3e, (((1,), (1,)), ((), ())),
                             preferred_element_type=jnp.float32)
    h = ((h1 * jax.nn.sigmoid(h1)) * h3).astype(jnp.bfloat16)
    ys_ref[...] = jax.lax.dot_general(h, w2e, (((1,), (1,)), ((), ())),
                                      preferred_element_type=jnp.float32)


TOKW = 128
SCW = 128
GW = 128
QTR = D_MODEL // 4


def _sc_mesh():
    return plsc.VectorSubcoreMesh(core_axis_name="core",
                                  subcore_axis_name="subcore")


def _scatter_tok(tokw, posflat):
    @pl.kernel(out_type=jax.ShapeDtypeStruct((M_PAD, TOKW), jnp.int32),
               mesh=_sc_mesh(), scratch_types=[])
    def k(tok_hbm, pos_hbm, o_hbm):
        def body(x_vmem, i_vmem):
            pltpu.sync_copy(x_vmem, o_hbm.at[i_vmem.at[0]])

        pltpu.emit_pipeline(
            body,
            grid=(M // SCW,),
            in_specs=[
                pl.BlockSpec((SCW, TOKW), lambda i: (i, 0)),
                pl.BlockSpec((1, SCW), lambda i: (0, i)),
            ],
            out_specs=[],
            core_axis_name=("core", "subcore"),
            dimension_semantics=(pltpu.PARALLEL,),
        )(tok_hbm, pos_hbm)

    return k(tokw, posflat)


def _sc_row_gather(src, idx):
    n = idx.shape[0]
    src4 = src.reshape(-1, QTR)
    idx4 = (4 * idx[:, None] + jnp.arange(4, dtype=jnp.int32)).reshape(
        1, 4 * n)
    n4 = 4 * n

    @pl.kernel(out_type=jax.ShapeDtypeStruct((n4, QTR), jnp.float32),
               mesh=_sc_mesh(), scratch_types=[])
    def k(src_hbm, i_hbm, o_hbm):
        def body(i_vmem, o_vmem):
            pltpu.sync_copy(src_hbm.at[i_vmem.at[0]], o_vmem)

        pltpu.emit_pipeline(
            body,
            grid=(n4 // GW,),
            in_specs=[pl.BlockSpec((1, GW), lambda i: (0, i))],
            out_specs=[pl.BlockSpec((GW, QTR), lambda i: (i, 0))],
            core_axis_name=("core", "subcore"),
            dimension_semantics=(pltpu.PARALLEL,),
        )(i_hbm, o_hbm)

    return k(src4, idx4).reshape(n, D_MODEL)


def _combine_kernel(yg_ref, wk_ref, o_ref):
    acc = jnp.zeros((o_ref.shape[0], D_MODEL), jnp.float32)
    for k in range(TOP_K):
        acc = acc + (wk_ref[:, k:k + 1]
                     * yg_ref[:, k * D_MODEL:(k + 1) * D_MODEL])
    o_ref[...] = acc


def kernel(hidden_states, layer_idx, gate_w, w1, w3, w2):
    del layer_idx
    i32 = jnp.int32

    pos, wk, counts2d = pl.pallas_call(
        _routing_kernel,
        out_shape=(
            jax.ShapeDtypeStruct((T, TOP_K), i32),
            jax.ShapeDtypeStruct((T, TOP_K), jnp.float32),
            jax.ShapeDtypeStruct((1, E), i32),
        ),
    )(hidden_states, gate_w)

    counts = counts2d[0]
    ptiles = (counts + (TM - 1)) // TM
    poffsets = jnp.concatenate(
        [jnp.zeros((1,), i32), jnp.cumsum(ptiles * TM, dtype=i32)])
    te = jnp.repeat(jnp.arange(E, dtype=i32), ptiles,
                    total_repeat_length=NT)
    n_tiles = jnp.sum(ptiles)
    te = jnp.where(jnp.arange(NT, dtype=i32) < n_tiles, te, E - 1)

    posflat = pos.reshape(1, M)

    tok8w = jnp.broadcast_to(
        jnp.repeat(jnp.arange(T, dtype=i32), TOP_K)[:, None], (M, TOKW))
    sorted_tokw = _scatter_tok(tok8w, posflat)
    e_of_slot = jnp.repeat(te, TM)
    slot = jnp.arange(M_PAD, dtype=i32)
    valid_slot = (slot - poffsets[e_of_slot]) < counts[e_of_slot]
    sorted_tok = jnp.where(valid_slot, sorted_tokw[:, 0], 0)
    xs = _sc_row_gather(hidden_states, sorted_tok)

    ys = pl.pallas_call(
        _gmm_kernel,
        grid_spec=pltpu.PrefetchScalarGridSpec(
            num_scalar_prefetch=1,
            grid=(NT,),
            in_specs=[
                pl.BlockSpec((TM, D_MODEL), lambda g, te: (g, 0)),
                pl.BlockSpec((1, D_FF, D_MODEL), lambda g, te: (te[g], 0, 0)),
                pl.BlockSpec((1, D_FF, D_MODEL), lambda g, te: (te[g], 0, 0)),
                pl.BlockSpec((1, D_MODEL, D_FF), lambda g, te: (te[g], 0, 0)),
            ],
            out_specs=pl.BlockSpec((TM, D_MODEL), lambda g, te: (g, 0)),
        ),
        out_shape=jax.ShapeDtypeStruct((M_PAD, D_MODEL), jnp.float32),
    )(te, xs, w1, w3, w2)

    yg = _sc_row_gather(ys, posflat[0])

    out = pl.pallas_call(
        _combine_kernel,
        grid=(8,),
        in_specs=[
            pl.BlockSpec((T // 8, TOP_K * D_MODEL), lambda i: (i, 0)),
            pl.BlockSpec((T // 8, TOP_K), lambda i: (i, 0)),
        ],
        out_specs=pl.BlockSpec((T // 8, D_MODEL), lambda i: (i, 0)),
        out_shape=jax.ShapeDtypeStruct((T, D_MODEL), jnp.float32),
    )(yg.reshape(T, TOP_K * D_MODEL), wk)
    return out


# speedup vs baseline: 1.3332x; 1.3332x over previous
import jax
import jax.numpy as jnp
from jax.experimental import pallas as pl
from jax.experimental.pallas import tpu as pltpu
from jax.experimental.pallas import tpu_sc as plsc

E = 64
TOP_K = 8
D_MODEL = 1024
D_FF = 512
N_GROUP = 8
TOPK_GROUP = 4
T = 2048
GS = E // N_GROUP
M = T * TOP_K
TM = 256
NT = M // TM + E
M_PAD = NT * TM


def _routing_kernel(x_ref, gw_ref, pos_ref, wk_ref, counts_ref):
    x = x_ref[...]
    gw = gw_ref[...]
    logits = jax.lax.dot_general(x, gw, (((1,), (1,)), ((), ())),
                                 preferred_element_type=jnp.float32)
    m = jnp.max(logits, axis=1, keepdims=True)
    ex = jnp.exp(logits - m)
    scores = ex / jnp.sum(ex, axis=1, keepdims=True)

    lane = jax.lax.broadcasted_iota(jnp.int32, (T, E), 1)
    group_of_lane = lane // GS

    G = jnp.zeros((T, E), jnp.float32)
    gmaxes = []
    for g in range(N_GROUP):
        gm = jnp.max(jnp.where(group_of_lane == g, scores, -jnp.inf), axis=1,
                     keepdims=True)
        gmaxes.append(gm)
        G = jnp.where(group_of_lane == g, gm, G)

    rank = jnp.zeros((T, E), jnp.int32)
    for g in range(N_GROUP):
        gm = gmaxes[g]
        rank = rank + jnp.where(gm > G, 1, 0) \
                    + jnp.where((gm == G) & (g < group_of_lane), 1, 0)
    ms = jnp.where(rank < TOPK_GROUP, scores, 0.0)

    work = ms
    denom = jnp.zeros((T, 1), jnp.float32)
    picks = []
    mxs = []
    for _ in range(TOP_K):
        mx = jnp.max(work, axis=1, keepdims=True)
        pick_lane = jnp.min(jnp.where(work == mx, lane, E), axis=1,
                            keepdims=True)
        pick = lane == pick_lane
        picks.append(pick)
        mxs.append(mx)
        denom = denom + mx
        work = jnp.where(pick, -1.0, work)

    chosen = picks[0]
    for p in picks[1:]:
        chosen = chosen | p
    c32 = chosen.astype(jnp.int32)

    inc = c32
    s = 1
    while s < T:
        shifted = jnp.concatenate(
            [jnp.zeros((s, E), jnp.int32), inc[: T - s, :]], axis=0)
        inc = inc + shifted
        s *= 2
    col_excl = inc - c32

    counts = jnp.sum(c32, axis=0, keepdims=True)
    counts_ref[...] = counts

    pc = ((counts + (TM - 1)) // TM) * TM
    off = pc
    s = 1
    while s < E:
        off_sh = jnp.concatenate(
            [jnp.zeros((1, s), jnp.int32), off[:, : E - s]], axis=1)
        off = off + off_sh
        s *= 2
    offsets = off - pc

    posmat = offsets + col_excl

    inv_denom = 1.0 / (denom + 1e-20)
    pos_cols = []
    wk_cols = []
    for k in range(TOP_K):
        pos_cols.append(jnp.sum(jnp.where(picks[k], posmat, 0), axis=1,
                                keepdims=True))
        wk_cols.append(mxs[k] * inv_denom)
    pos_ref[...] = jnp.concatenate(pos_cols, axis=1)
    wk_ref[...] = jnp.concatenate(wk_cols, axis=1)


def _gmm_kernel(te_ref, xs_ref, w1_ref, w3_ref, w2_ref, ys_ref):
    x = xs_ref[...].astype(jnp.bfloat16)
    w1e = w1_ref[0].astype(jnp.bfloat16)
    w3e = w3_ref[0].astype(jnp.bfloat16)
    w2e = w2_ref[0].astype(jnp.bfloat16)
    h1 = jax.lax.dot_general(x, w1e, (((1,), (1,)), ((), ())),
                             preferred_element_type=jnp.float32)
    h3 = jax.lax.dot_general(x, w3e, (((1,), (1,)), ((), ())),
                             preferred_element_type=jnp.float32)
    h = ((h1 * jax.nn.sigmoid(h1)) * h3).astype(jnp.bfloat16)
    ys_ref[...] = jax.lax.dot_general(h, w2e, (((1,), (1,)), ((), ())),
                                      preferred_element_type=jnp.float32)


TOKW = 128
SCW = 128
GW = 128
QTR = D_MODEL // 4


def _sc_mesh():
    return plsc.VectorSubcoreMesh(core_axis_name="core",
                                  subcore_axis_name="subcore")


def _scatter_tok(tokw, posflat):
    @pl.kernel(out_type=jax.ShapeDtypeStruct((M_PAD, TOKW), jnp.int32),
               mesh=_sc_mesh(), scratch_types=[])
    def k(tok_hbm, pos_hbm, o_hbm):
        def body(x_vmem, i_vmem):
            pltpu.sync_copy(x_vmem, o_hbm.at[i_vmem.at[0]])

        pltpu.emit_pipeline(
            body,
            grid=(M // SCW,),
            in_specs=[
                pl.BlockSpec((SCW, TOKW), lambda i: (i, 0)),
                pl.BlockSpec((1, SCW), lambda i: (0, i)),
            ],
            out_specs=[],
            core_axis_name=("core", "subcore"),
            dimension_semantics=(pltpu.PARALLEL,),
        )(tok_hbm, pos_hbm)

    return k(tokw, posflat)


def _sc_row_gather(src, idx):
    n = idx.shape[0]
    src4 = src.reshape(-1, QTR)
    idx4 = (4 * idx[:, None] + jnp.arange(4, dtype=jnp.int32)).reshape(
        1, 4 * n)
    n4 = 4 * n

    @pl.kernel(out_type=jax.ShapeDtypeStruct((n4, QTR), jnp.float32),
               mesh=_sc_mesh(), scratch_types=[])
    def k(src_hbm, i_hbm, o_hbm):
        def body(i_vmem, o_vmem):
            pltpu.sync_copy(src_hbm.at[i_vmem.at[0]], o_vmem)

        pltpu.emit_pipeline(
            body,
            grid=(n4 // GW,),
            in_specs=[pl.BlockSpec((1, GW), lambda i: (0, i))],
            out_specs=[pl.BlockSpec((GW, QTR), lambda i: (i, 0))],
            core_axis_name=("core", "subcore"),
            dimension_semantics=(pltpu.PARALLEL,),
        )(i_hbm, o_hbm)

    return k(src4, idx4).reshape(n, D_MODEL)


def _combine_kernel(yg_ref, wk_ref, o_ref):
    acc = jnp.zeros((o_ref.shape[0], D_MODEL), jnp.float32)
    for k in range(TOP_K):
        acc = acc + (wk_ref[:, k:k + 1]
                     * yg_ref[:, k * D_MODEL:(k + 1) * D_MODEL])
    o_ref[...] = acc


def kernel(hidden_states, layer_idx, gate_w, w1, w3, w2):
    del layer_idx
    i32 = jnp.int32

    pos, wk, counts2d = pl.pallas_call(
        _routing_kernel,
        out_shape=(
            jax.ShapeDtypeStruct((T, TOP_K), i32),
            jax.ShapeDtypeStruct((T, TOP_K), jnp.float32),
            jax.ShapeDtypeStruct((1, E), i32),
        ),
    )(hidden_states, gate_w)

    counts = counts2d[0]
    ptiles = (counts + (TM - 1)) // TM
    poffsets = jnp.concatenate(
        [jnp.zeros((1,), i32), jnp.cumsum(ptiles * TM, dtype=i32)])
    te = jnp.repeat(jnp.arange(E, dtype=i32), ptiles,
                    total_repeat_length=NT)
    n_tiles = jnp.sum(ptiles)
    te = jnp.where(jnp.arange(NT, dtype=i32) < n_tiles, te, E - 1)

    posflat = pos.reshape(1, M)

    tok8w = jnp.broadcast_to(
        jnp.repeat(jnp.arange(T, dtype=i32), TOP_K)[:, None], (M, TOKW))
    sorted_tokw = _scatter_tok(tok8w, posflat)
    poff_slot = jnp.repeat(poffsets[te], TM)
    cnt_slot = jnp.repeat(counts[te], TM)
    slot = jnp.arange(M_PAD, dtype=i32)
    valid_slot = (slot - poff_slot) < cnt_slot
    sorted_tok = jnp.where(valid_slot, sorted_tokw[:, 0], 0)
    xs = _sc_row_gather(hidden_states, sorted_tok)

    ys = pl.pallas_call(
        _gmm_kernel,
        grid_spec=pltpu.PrefetchScalarGridSpec(
            num_scalar_prefetch=1,
            grid=(NT,),
            in_specs=[
                pl.BlockSpec((TM, D_MODEL), lambda g, te: (g, 0)),
                pl.BlockSpec((1, D_FF, D_MODEL), lambda g, te: (te[g], 0, 0)),
                pl.BlockSpec((1, D_FF, D_MODEL), lambda g, te: (te[g], 0, 0)),
                pl.BlockSpec((1, D_MODEL, D_FF), lambda g, te: (te[g], 0, 0)),
            ],
            out_specs=pl.BlockSpec((TM, D_MODEL), lambda g, te: (g, 0)),
        ),
        out_shape=jax.ShapeDtypeStruct((M_PAD, D_MODEL), jnp.float32),
    )(te, xs, w1, w3, w2)

    yg = _sc_row_gather(ys, posflat[0])

    out = pl.pallas_call(
        _combine_kernel,
        grid=(8,),
        in_specs=[
            pl.BlockSpec((T // 8, TOP_K * D_MODEL), lambda i: (i, 0)),
            pl.BlockSpec((T // 8, TOP_K), lambda i: (i, 0)),
        ],
        out_specs=pl.BlockSpec((T // 8, D_MODEL), lambda i: (i, 0)),
        out_shape=jax.ShapeDtypeStruct((T, D_MODEL), jnp.float32),
    )(yg.reshape(T, TOP_K * D_MODEL), wk)
    return out


# speedup vs baseline: 2.3639x; 1.7732x over previous
import jax
import jax.numpy as jnp
from jax.experimental import pallas as pl
from jax.experimental.pallas import tpu as pltpu
from jax.experimental.pallas import tpu_sc as plsc

E = 64
TOP_K = 8
D_MODEL = 1024
D_FF = 512
N_GROUP = 8
TOPK_GROUP = 4
T = 2048
GS = E // N_GROUP
M = T * TOP_K
TM = 256
NT = M // TM + E
M_PAD = NT * TM


def _routing_kernel(x_ref, gw_ref, pos_ref, wk_ref, counts_ref):
    x = x_ref[...]
    gw = gw_ref[...]
    logits = jax.lax.dot_general(x, gw, (((1,), (1,)), ((), ())),
                                 preferred_element_type=jnp.float32)
    m = jnp.max(logits, axis=1, keepdims=True)
    ex = jnp.exp(logits - m)
    scores = ex / jnp.sum(ex, axis=1, keepdims=True)

    lane = jax.lax.broadcasted_iota(jnp.int32, (T, E), 1)
    group_of_lane = lane // GS

    G = jnp.zeros((T, E), jnp.float32)
    gmaxes = []
    for g in range(N_GROUP):
        gm = jnp.max(jnp.where(group_of_lane == g, scores, -jnp.inf), axis=1,
                     keepdims=True)
        gmaxes.append(gm)
        G = jnp.where(group_of_lane == g, gm, G)

    rank = jnp.zeros((T, E), jnp.int32)
    for g in range(N_GROUP):
        gm = gmaxes[g]
        rank = rank + jnp.where(gm > G, 1, 0) \
                    + jnp.where((gm == G) & (g < group_of_lane), 1, 0)
    ms = jnp.where(rank < TOPK_GROUP, scores, 0.0)

    work = ms
    denom = jnp.zeros((T, 1), jnp.float32)
    picks = []
    mxs = []
    for _ in range(TOP_K):
        mx = jnp.max(work, axis=1, keepdims=True)
        pick_lane = jnp.min(jnp.where(work == mx, lane, E), axis=1,
                            keepdims=True)
        pick = lane == pick_lane
        picks.append(pick)
        mxs.append(mx)
        denom = denom + mx
        work = jnp.where(pick, -1.0, work)

    chosen = picks[0]
    for p in picks[1:]:
        chosen = chosen | p
    c32 = chosen.astype(jnp.int32)

    inc = c32
    s = 1
    while s < T:
        shifted = jnp.concatenate(
            [jnp.zeros((s, E), jnp.int32), inc[: T - s, :]], axis=0)
        inc = inc + shifted
        s *= 2
    col_excl = inc - c32

    counts = jnp.sum(c32, axis=0, keepdims=True)
    counts_ref[...] = counts

    pc = ((counts + (TM - 1)) // TM) * TM
    off = pc
    s = 1
    while s < E:
        off_sh = jnp.concatenate(
            [jnp.zeros((1, s), jnp.int32), off[:, : E - s]], axis=1)
        off = off + off_sh
        s *= 2
    offsets = off - pc

    posmat = offsets + col_excl

    inv_denom = 1.0 / (denom + 1e-20)
    pos_cols = []
    wk_cols = []
    for k in range(TOP_K):
        pos_cols.append(jnp.sum(jnp.where(picks[k], posmat, 0), axis=1,
                                keepdims=True))
        wk_cols.append(mxs[k] * inv_denom)
    pos_ref[...] = jnp.concatenate(pos_cols, axis=1)
    wk_ref[...] = jnp.concatenate(wk_cols, axis=1)


def _gmm_kernel(te_ref, xs_ref, w1_ref, w3_ref, w2_ref, ys_ref):
    x = xs_ref[...].astype(jnp.bfloat16)
    w1e = w1_ref[0].astype(jnp.bfloat16)
    w3e = w3_ref[0].astype(jnp.bfloat16)
    w2e = w2_ref[0].astype(jnp.bfloat16)
    h1 = jax.lax.dot_general(x, w1e, (((1,), (1,)), ((), ())),
                             preferred_element_type=jnp.float32)
    h3 = jax.lax.dot_general(x, w3e, (((1,), (1,)), ((), ())),
                             preferred_element_type=jnp.float32)
    h = ((h1 * jax.nn.sigmoid(h1)) * h3).astype(jnp.bfloat16)
    ys_ref[...] = jax.lax.dot_general(h, w2e, (((1,), (1,)), ((), ())),
                                      preferred_element_type=jnp.float32)


TOKW = 128
SCW = 128
GW = 128
QTR = D_MODEL // 4


def _sc_mesh():
    return plsc.VectorSubcoreMesh(core_axis_name="core",
                                  subcore_axis_name="subcore")


def _scatter_tok(tokw, posflat):
    @pl.kernel(out_type=jax.ShapeDtypeStruct((M_PAD, TOKW), jnp.int32),
               mesh=_sc_mesh(), scratch_types=[])
    def k(tok_hbm, pos_hbm, o_hbm):
        def body(x_vmem, i_vmem):
            pltpu.sync_copy(x_vmem, o_hbm.at[i_vmem.at[0]])

        pltpu.emit_pipeline(
            body,
            grid=(M // SCW,),
            in_specs=[
                pl.BlockSpec((SCW, TOKW), lambda i: (i, 0)),
                pl.BlockSpec((1, SCW), lambda i: (0, i)),
            ],
            out_specs=[],
            core_axis_name=("core", "subcore"),
            dimension_semantics=(pltpu.PARALLEL,),
        )(tok_hbm, pos_hbm)

    return k(tokw, posflat)


def _sc_row_gather(src, idx):
    n = idx.shape[0]
    src4 = src.reshape(-1, QTR)
    idx4 = (4 * idx[:, None] + jnp.arange(4, dtype=jnp.int32)).reshape(
        1, 4 * n)
    n4 = 4 * n

    @pl.kernel(out_type=jax.ShapeDtypeStruct((n4, QTR), jnp.float32),
               mesh=_sc_mesh(), scratch_types=[])
    def k(src_hbm, i_hbm, o_hbm):
        def body(i_vmem, o_vmem):
            pltpu.sync_copy(src_hbm.at[i_vmem.at[0]], o_vmem)

        pltpu.emit_pipeline(
            body,
            grid=(n4 // GW,),
            in_specs=[pl.BlockSpec((1, GW), lambda i: (0, i))],
            out_specs=[pl.BlockSpec((GW, QTR), lambda i: (i, 0))],
            core_axis_name=("core", "subcore"),
            dimension_semantics=(pltpu.PARALLEL,),
        )(i_hbm, o_hbm)

    return k(src4, idx4).reshape(n, D_MODEL)


def _combine_kernel(yg_ref, wk_ref, o_ref):
    acc = jnp.zeros((o_ref.shape[0], D_MODEL), jnp.float32)
    for k in range(TOP_K):
        acc = acc + (wk_ref[:, k:k + 1]
                     * yg_ref[:, k * D_MODEL:(k + 1) * D_MODEL])
    o_ref[...] = acc


def kernel(hidden_states, layer_idx, gate_w, w1, w3, w2):
    del layer_idx
    i32 = jnp.int32

    pos, wk, counts2d = pl.pallas_call(
        _routing_kernel,
        out_shape=(
            jax.ShapeDtypeStruct((T, TOP_K), i32),
            jax.ShapeDtypeStruct((T, TOP_K), jnp.float32),
            jax.ShapeDtypeStruct((1, E), i32),
        ),
    )(hidden_states, gate_w)

    counts = counts2d[0]
    ptiles = (counts + (TM - 1)) // TM
    poffsets = jnp.concatenate(
        [jnp.zeros((1,), i32), jnp.cumsum(ptiles * TM, dtype=i32)])
    te = jnp.repeat(jnp.arange(E, dtype=i32), ptiles,
                    total_repeat_length=NT)
    n_tiles = jnp.sum(ptiles)
    te = jnp.where(jnp.arange(NT, dtype=i32) < n_tiles, te, E - 1)

    posflat = pos.reshape(1, M)

    tok8w = jnp.broadcast_to(
        jnp.repeat(jnp.arange(T, dtype=i32), TOP_K)[:, None], (M, TOKW))
    sorted_tokw = _scatter_tok(tok8w, posflat)
    poff_slot = jnp.repeat(poffsets[te], TM)
    cnt_slot = jnp.repeat(counts[te], TM)
    slot = jnp.arange(M_PAD, dtype=i32)
    valid_slot = (slot - poff_slot) < cnt_slot
    sorted_tok = jnp.where(valid_slot, sorted_tokw[:, 0], slot % T)
    xs = _sc_row_gather(hidden_states, sorted_tok)

    ys = pl.pallas_call(
        _gmm_kernel,
        grid_spec=pltpu.PrefetchScalarGridSpec(
            num_scalar_prefetch=1,
            grid=(NT,),
            in_specs=[
                pl.BlockSpec((TM, D_MODEL), lambda g, te: (g, 0)),
                pl.BlockSpec((1, D_FF, D_MODEL), lambda g, te: (te[g], 0, 0)),
                pl.BlockSpec((1, D_FF, D_MODEL), lambda g, te: (te[g], 0, 0)),
                pl.BlockSpec((1, D_MODEL, D_FF), lambda g, te: (te[g], 0, 0)),
            ],
            out_specs=pl.BlockSpec((TM, D_MODEL), lambda g, te: (g, 0)),
        ),
        out_shape=jax.ShapeDtypeStruct((M_PAD, D_MODEL), jnp.float32),
    )(te, xs, w1, w3, w2)

    yg = _sc_row_gather(ys, posflat[0])

    out = pl.pallas_call(
        _combine_kernel,
        grid=(8,),
        in_specs=[
            pl.BlockSpec((T // 8, TOP_K * D_MODEL), lambda i: (i, 0)),
            pl.BlockSpec((T // 8, TOP_K), lambda i: (i, 0)),
        ],
        out_specs=pl.BlockSpec((T // 8, D_MODEL), lambda i: (i, 0)),
        out_shape=jax.ShapeDtypeStruct((T, D_MODEL), jnp.float32),
    )(yg.reshape(T, TOP_K * D_MODEL), wk)
    return out


# speedup vs baseline: 3.2435x; 1.3721x over previous
import jax
import jax.numpy as jnp
from jax.experimental import pallas as pl
from jax.experimental.pallas import tpu as pltpu
from jax.experimental.pallas import tpu_sc as plsc

E = 64
TOP_K = 8
D_MODEL = 1024
D_FF = 512
N_GROUP = 8
TOPK_GROUP = 4
T = 2048
GS = E // N_GROUP
M = T * TOP_K
TM = 512
NT = M // TM
NV = NT + E - 1


def _routing_kernel(x_ref, gw_ref, pos_ref, wk_ref, counts_ref):
    x = x_ref[...]
    gw = gw_ref[...]
    logits = jax.lax.dot_general(x, gw, (((1,), (1,)), ((), ())),
                                 preferred_element_type=jnp.float32)
    m = jnp.max(logits, axis=1, keepdims=True)
    ex = jnp.exp(logits - m)
    scores = ex / jnp.sum(ex, axis=1, keepdims=True)

    lane = jax.lax.broadcasted_iota(jnp.int32, (T, E), 1)
    group_of_lane = lane // GS

    G = jnp.zeros((T, E), jnp.float32)
    gmaxes = []
    for g in range(N_GROUP):
        gm = jnp.max(jnp.where(group_of_lane == g, scores, -jnp.inf), axis=1,
                     keepdims=True)
        gmaxes.append(gm)
        G = jnp.where(group_of_lane == g, gm, G)

    rank = jnp.zeros((T, E), jnp.int32)
    for g in range(N_GROUP):
        gm = gmaxes[g]
        rank = rank + jnp.where(gm > G, 1, 0) \
                    + jnp.where((gm == G) & (g < group_of_lane), 1, 0)
    ms = jnp.where(rank < TOPK_GROUP, scores, 0.0)

    work = ms
    denom = jnp.zeros((T, 1), jnp.float32)
    picks = []
    mxs = []
    for _ in range(TOP_K):
        mx = jnp.max(work, axis=1, keepdims=True)
        pick_lane = jnp.min(jnp.where(work == mx, lane, E), axis=1,
                            keepdims=True)
        pick = lane == pick_lane
        picks.append(pick)
        mxs.append(mx)
        denom = denom + mx
        work = jnp.where(pick, -1.0, work)

    chosen = picks[0]
    for p in picks[1:]:
        chosen = chosen | p
    c32 = chosen.astype(jnp.int32)

    inc = c32
    s = 1
    while s < T:
        shifted = jnp.concatenate(
            [jnp.zeros((s, E), jnp.int32), inc[: T - s, :]], axis=0)
        inc = inc + shifted
        s *= 2
    col_excl = inc - c32

    counts = jnp.sum(c32, axis=0, keepdims=True)
    counts_ref[...] = counts

    off = counts
    s = 1
    while s < E:
        off_sh = jnp.concatenate(
            [jnp.zeros((1, s), jnp.int32), off[:, : E - s]], axis=1)
        off = off + off_sh
        s *= 2
    offsets = off - counts

    posmat = offsets + col_excl

    inv_denom = 1.0 / (denom + 1e-20)
    pos_cols = []
    wk_cols = []
    for k in range(TOP_K):
        pos_cols.append(jnp.sum(jnp.where(picks[k], posmat, 0), axis=1,
                                keepdims=True))
        wk_cols.append(mxs[k] * inv_denom)
    pos_ref[...] = jnp.concatenate(pos_cols, axis=1)
    wk_ref[...] = jnp.concatenate(wk_cols, axis=1)


def _gmm_kernel(vt_ref, ve_ref, lo_ref, hi_ref, fr_ref,
                xs_ref, w1_ref, w3_ref, w2_ref, ys_ref):
    v = pl.program_id(0)
    lo = lo_ref[v]
    hi = hi_ref[v]
    row0 = vt_ref[v] * TM
    rows = row0 + jax.lax.broadcasted_iota(jnp.int32, (TM, 1), 0)
    mask = (rows >= lo) & (rows < hi)

    x = xs_ref[...].astype(jnp.bfloat16)
    w1e = w1_ref[0].astype(jnp.bfloat16)
    w3e = w3_ref[0].astype(jnp.bfloat16)
    w2e = w2_ref[0].astype(jnp.bfloat16)
    h1 = jax.lax.dot_general(x, w1e, (((1,), (1,)), ((), ())),
                             preferred_element_type=jnp.float32)
    h3 = jax.lax.dot_general(x, w3e, (((1,), (1,)), ((), ())),
                             preferred_element_type=jnp.float32)
    h = ((h1 * jax.nn.sigmoid(h1)) * h3).astype(jnp.bfloat16)
    y = jax.lax.dot_general(h, w2e, (((1,), (1,)), ((), ())),
                            preferred_element_type=jnp.float32)
    contrib = jnp.where(mask, y, 0.0)

    @pl.when(fr_ref[v] == 1)
    def _():
        ys_ref[...] = contrib

    @pl.when(fr_ref[v] == 0)
    def _():
        ys_ref[...] = ys_ref[...] + contrib


TOKW = 128
SCW = 128
GW = 128
QTR = D_MODEL // 4


def _sc_mesh():
    return plsc.VectorSubcoreMesh(core_axis_name="core",
                                  subcore_axis_name="subcore")


def _scatter_tok(tokw, posflat):
    @pl.kernel(out_type=jax.ShapeDtypeStruct((M, TOKW), jnp.int32),
               mesh=_sc_mesh(), scratch_types=[])
    def k(tok_hbm, pos_hbm, o_hbm):
        def body(x_vmem, i_vmem):
            pltpu.sync_copy(x_vmem, o_hbm.at[i_vmem.at[0]])

        pltpu.emit_pipeline(
            body,
            grid=(M // SCW,),
            in_specs=[
                pl.BlockSpec((SCW, TOKW), lambda i: (i, 0)),
                pl.BlockSpec((1, SCW), lambda i: (0, i)),
            ],
            out_specs=[],
            core_axis_name=("core", "subcore"),
            dimension_semantics=(pltpu.PARALLEL,),
        )(tok_hbm, pos_hbm)

    return k(tokw, posflat)


def _sc_row_gather(src, idx):
    src4 = src.reshape(-1, QTR)
    idx4 = (4 * idx[:, None] + jnp.arange(4, dtype=jnp.int32)).reshape(
        1, 4 * M)
    n4 = 4 * M

    @pl.kernel(out_type=jax.ShapeDtypeStruct((n4, QTR), jnp.float32),
               mesh=_sc_mesh(), scratch_types=[])
    def k(src_hbm, i_hbm, o_hbm):
        def body(i_vmem, o_vmem):
            pltpu.sync_copy(src_hbm.at[i_vmem.at[0]], o_vmem)

        pltpu.emit_pipeline(
            body,
            grid=(n4 // GW,),
            in_specs=[pl.BlockSpec((1, GW), lambda i: (0, i))],
            out_specs=[pl.BlockSpec((GW, QTR), lambda i: (i, 0))],
            core_axis_name=("core", "subcore"),
            dimension_semantics=(pltpu.PARALLEL,),
        )(i_hbm, o_hbm)

    return k(src4, idx4).reshape(M, D_MODEL)


def _combine_kernel(yg_ref, wk_ref, o_ref):
    acc = jnp.zeros((o_ref.shape[0], D_MODEL), jnp.float32)
    for k in range(TOP_K):
        acc = acc + (wk_ref[:, k:k + 1]
                     * yg_ref[:, k * D_MODEL:(k + 1) * D_MODEL])
    o_ref[...] = acc


def kernel(hidden_states, layer_idx, gate_w, w1, w3, w2):
    del layer_idx
    i32 = jnp.int32

    pos, wk, counts2d = pl.pallas_call(
        _routing_kernel,
        out_shape=(
            jax.ShapeDtypeStruct((T, TOP_K), i32),
            jax.ShapeDtypeStruct((T, TOP_K), jnp.float32),
            jax.ShapeDtypeStruct((1, E), i32),
        ),
    )(hidden_states, gate_w)

    counts = counts2d[0]
    offsets = jnp.concatenate(
        [jnp.zeros((1,), i32), jnp.cumsum(counts, dtype=i32)])
    f_e = offsets[:E] // TM
    l_e = jnp.maximum(offsets[1:] - 1, 0) // TM
    nv_e = jnp.where(counts > 0, l_e - f_e + 1, 0)
    n_vis = jnp.sum(nv_e)
    start_vis = jnp.cumsum(nv_e) - nv_e
    vidx = jnp.arange(NV, dtype=i32)
    valid = vidx < n_vis
    ve = jnp.where(valid, jnp.repeat(jnp.arange(E, dtype=i32), nv_e,
                                     total_repeat_length=NV), E - 1)
    vt = jnp.where(valid, f_e[ve] + (vidx - start_vis[ve]), NT - 1)
    vt = jnp.clip(vt, 0, NT - 1).astype(i32)
    lo = jnp.where(valid, offsets[ve], 0).astype(i32)
    hi = jnp.where(valid, offsets[ve + 1], 0).astype(i32)
    fr = (jnp.concatenate([jnp.ones((1,), jnp.bool_), vt[1:] != vt[:-1]])
          & valid).astype(i32)

    posflat = pos.reshape(1, M)

    tok8w = jnp.broadcast_to(
        jnp.repeat(jnp.arange(T, dtype=i32), TOP_K)[:, None], (M, TOKW))
    sorted_tokw = _scatter_tok(tok8w, posflat)
    sorted_tok = sorted_tokw[:, 0]
    xs = _sc_row_gather(hidden_states, sorted_tok)

    ys = pl.pallas_call(
        _gmm_kernel,
        grid_spec=pltpu.PrefetchScalarGridSpec(
            num_scalar_prefetch=5,
            grid=(NV,),
            in_specs=[
                pl.BlockSpec((TM, D_MODEL),
                             lambda v, vt, ve, lo, hi, fr: (vt[v], 0)),
                pl.BlockSpec((1, D_FF, D_MODEL),
                             lambda v, vt, ve, lo, hi, fr: (ve[v], 0, 0)),
                pl.BlockSpec((1, D_FF, D_MODEL),
                             lambda v, vt, ve, lo, hi, fr: (ve[v], 0, 0)),
                pl.BlockSpec((1, D_MODEL, D_FF),
                             lambda v, vt, ve, lo, hi, fr: (ve[v], 0, 0)),
            ],
            out_specs=pl.BlockSpec((TM, D_MODEL),
                                   lambda v, vt, ve, lo, hi, fr: (vt[v], 0)),
        ),
        out_shape=jax.ShapeDtypeStruct((M, D_MODEL), jnp.float32),
    )(vt, ve, lo, hi, fr, xs, w1, w3, w2)

    yg = _sc_row_gather(ys, posflat[0])

    out = pl.pallas_call(
        _combine_kernel,
        grid=(8,),
        in_specs=[
            pl.BlockSpec((T // 8, TOP_K * D_MODEL), lambda i: (i, 0)),
            pl.BlockSpec((T // 8, TOP_K), lambda i: (i, 0)),
        ],
        out_specs=pl.BlockSpec((T // 8, D_MODEL), lambda i: (i, 0)),
        out_shape=jax.ShapeDtypeStruct((T, D_MODEL), jnp.float32),
    )(yg.reshape(T, TOP_K * D_MODEL), wk)
    return out
